# baseline (speedup 1.0000x reference)
import jax
import jax.numpy as jnp
from jax.experimental import pallas as pl

_ANCHORS = [1.3221, 1.73145, 3.19275, 4.00944, 5.05587, 8.09892, 9.47112,
            4.84053, 11.2364, 10.0071]
_NA = 5
_W = 52
_H = 52
_HW = _H * _W
_G = 16
_R = _G * _NA


def _prep_kernel(t_ref, p_ref):
    t = t_ref[...]
    gx = t[:, 0:1] * _W
    gy = t[:, 1:2] * _H
    gw = t[:, 2:3] * _W
    gh = t[:, 3:4] * _H
    gif = jnp.floor(gx)
    gjf = jnp.floor(gy)
    garea = gw * gh
    best_iou = jnp.full_like(gx, -1.0)
    best = jnp.zeros_like(gx)
    awb = jnp.zeros_like(gx)
    ahb = jnp.zeros_like(gx)
    for a in range(_NA):
        aw = _ANCHORS[2 * a]
        ah = _ANCHORS[2 * a + 1]
        cw = jnp.minimum(gw, aw)
        ch = jnp.minimum(gh, ah)
        carea = cw * ch
        iou = carea / (garea + aw * ah - carea)
        upd = iou > best_iou
        best = jnp.where(upd, float(a), best)
        awb = jnp.where(upd, aw, awb)
        ahb = jnp.where(upd, ah, ahb)
        best_iou = jnp.where(upd, iou, best_iou)
    tx = gx - gif
    ty = gy - gjf
    tw = jnp.log(gw / awb)
    th = jnp.log(gh / ahb)
    kmatch = gjf * float(_W) + gif
    p_ref[...] = jnp.concatenate(
        [gx, gy, gw, gh, gif, gjf, tx, ty, tw, th, awb, ahb, kmatch, best,
         garea, jnp.zeros_like(gx)], axis=1)


def _dense_kernel(p_ref, x_ref, out_ref):
    g = pl.program_id(0)

    @pl.when(g == 0)
    def _init():
        out_ref[0:1, 0:1] = jnp.zeros((1, 1), jnp.float32)

    x4 = x_ref[...].reshape(_R, _NA, _HW)
    X = x4[:, 0, :]
    Y = x4[:, 1, :]
    Wc = x4[:, 2, :]
    Hc = x4[:, 3, :]
    C = x4[:, 4, :]

    rio = jax.lax.broadcasted_iota(jnp.int32, (_R, _G), 0) // _NA
    cio = jax.lax.broadcasted_iota(jnp.int32, (_R, _G), 1)
    E = (rio == cio).astype(jnp.float32)
    EP = jnp.dot(E, p_ref[...], preferred_element_type=jnp.float32)

    def col(i):
        return EP[:, i:i + 1]

    gx, gy, gw, gh = col(0), col(1), col(2), col(3)
    gif, gjf = col(4), col(5)
    tx, ty, tw, th = col(6), col(7), col(8), col(9)
    kmatch, best, garea = col(12), col(13), col(14)

    aidx = (jax.lax.broadcasted_iota(jnp.int32, (_R, 1), 0) % _NA
            ).astype(jnp.float32)
    anw = jnp.zeros((_R, 1), jnp.float32)
    anh = jnp.zeros((_R, 1), jnp.float32)
    for a in range(_NA):
        m = aidx == float(a)
        anw = jnp.where(m, _ANCHORS[2 * a], anw)
        anh = jnp.where(m, _ANCHORS[2 * a + 1], anh)

    kio = jax.lax.broadcasted_iota(jnp.int32, (_R, _HW), 1)
    gyi = kio // _W
    gridx = (kio - gyi * _W).astype(jnp.float32)
    gridy = gyi.astype(jnp.float32)
    kiof = kio.astype(jnp.float32)

    gx0 = gx - 0.5 * gw
    gx1 = gx + 0.5 * gw
    gy0 = gy - 0.5 * gh
    gy1 = gy + 0.5 * gh

    sigx = jax.nn.sigmoid(X)
    sigy = jax.nn.sigmoid(Y)
    pc = jax.nn.sigmoid(C)
    pwv = jnp.exp(Wc) * anw
    phv = jnp.exp(Hc) * anh
    pxv = sigx + gridx
    pyv = sigy + gridy
    hw_ = 0.5 * pwv
    hh_ = 0.5 * phv
    uw = jnp.maximum(gx1, pxv + hw_) - jnp.minimum(gx0, pxv - hw_)
    uh = jnp.maximum(gy1, pyv + hh_) - jnp.minimum(gy0, pyv - hh_)
    cw = gw + pwv - uw
    ch = gh + phv - uh
    carea = cw * ch
    uarea = garea + pwv * phv - carea
    sil = (cw > 0.0) & (ch > 0.0) & (carea > 0.6 * uarea)
    dx = sigx - 0.5
    dy = sigy - 0.5
    cell = dx * dx + dy * dy + Wc * Wc + Hc * Hc \
        + jnp.where(sil, 0.0, pc * pc)
    base = jnp.sum(cell, axis=1, keepdims=True)

    rowm = best == aidx
    sel = rowm & (kiof == kmatch)
    r0 = jnp.sum(jnp.where(sel, X, 0.0), axis=1, keepdims=True)
    r1 = jnp.sum(jnp.where(sel, Y, 0.0), axis=1, keepdims=True)
    r2 = jnp.sum(jnp.where(sel, Wc, 0.0), axis=1, keepdims=True)
    r3 = jnp.sum(jnp.where(sel, Hc, 0.0), axis=1, keepdims=True)
    r4 = jnp.sum(jnp.where(sel, C, 0.0), axis=1, keepdims=True)

    sxm = jax.nn.sigmoid(r0)
    sym = jax.nn.sigmoid(r1)
    pcm = jax.nn.sigmoid(r4)
    pwm = jnp.exp(r2) * anw
    phm = jnp.exp(r3) * anh
    pxm = sxm + gif
    pym = sym + gjf
    uwm = jnp.maximum(gx1, pxm + 0.5 * pwm) - jnp.minimum(gx0, pxm - 0.5 * pwm)
    uhm = jnp.maximum(gy1, pym + 0.5 * phm) - jnp.minimum(gy0, pym - 0.5 * phm)
    cwm = gw + pwm - uwm
    chm = gh + phm - uhm
    cam = cwm * chm
    uam = garea + pwm * phm - cam
    iou_t = jnp.where((cwm > 0.0) & (chm > 0.0), cam / uam, 0.0)

    coord_corr = (sxm - tx) ** 2 - (sxm - 0.5) ** 2 \
        + (sym - ty) ** 2 - (sym - 0.5) ** 2 \
        + (r2 - tw) ** 2 - r2 * r2 \
        + (r3 - th) ** 2 - r3 * r3
    dconf = pcm - iou_t
    conf_corr = 5.0 * dconf * dconf \
        - jnp.where(iou_t > 0.6, 0.0, pcm * pcm)
    corr = jnp.where(rowm, coord_corr + conf_corr, 0.0)

    step = jnp.sum(base + corr, axis=0, keepdims=True)
    out_ref[0:1, 0:1] += step[0:1, 0:1]


def kernel(pred, target):
    bs = pred.shape[0]
    pred3 = pred.reshape(bs, _NA * 5, _HW)
    params = pl.pallas_call(
        _prep_kernel,
        out_shape=jax.ShapeDtypeStruct((bs, 16), jnp.float32),
    )(target)
    total = pl.pallas_call(
        _dense_kernel,
        grid=(bs // _G,),
        in_specs=[
            pl.BlockSpec((_G, 16), lambda g: (g, 0)),
            pl.BlockSpec((_G, _NA * 5, _HW), lambda g: (g, 0, 0)),
        ],
        out_specs=pl.BlockSpec((1, 1), lambda g: (0, 0)),
        out_shape=jax.ShapeDtypeStruct((1, 1), jnp.float32),
    )(params, pred3)
    return total[0, 0] * 0.5
